# baseline (speedup 1.0000x reference)
import numpy as np

import jax
import jax.numpy as jnp
from jax.experimental import pallas as pl

BATCH = 32
SEQ = 64
IN_W = 263
IN_WP = 384
CH = 512
CBD = 10
NCODE = 1024
F32 = jnp.float32


def _mm(a, b):
    return jax.lax.dot_general(a, b, (((1,), (0,)), ((), ())),
                               preferred_element_type=F32,
                               precision=jax.lax.Precision.DEFAULT)


def _shift_up(x, r):
    return jnp.concatenate([x[r:], jnp.zeros((r, x.shape[1]), F32)], axis=0)


def _shift_dn(x, r):
    return jnp.concatenate([jnp.zeros((r, x.shape[1]), F32), x[:-r]], axis=0)


def _conv3(x, wcat_ref, b_ref, dil):
    r = dil * BATCH
    rows, cin = x.shape
    if r >= rows:
        y = _mm(x, wcat_ref[cin:2 * cin])
    else:
        xcat = jnp.concatenate([_shift_dn(x, r), x, _shift_up(x, r)], axis=1)
        y = _mm(xcat, wcat_ref[...])
    return y + b_ref[...]


def _res_block(x, w1_ref, b1_ref, w2_ref, b2_ref, dil):
    h = jnp.maximum(x, 0.0)
    h = _conv3(h, w1_ref, b1_ref, dil)
    h = jnp.maximum(h, 0.0)
    h = _mm(h, w2_ref[...]) + b2_ref[...]
    return x + h


def _down_conv(he, ho, wcat_ref, b_ref):
    xcat = jnp.concatenate(
        [_shift_dn(ho, BATCH), he, ho, _shift_up(he, BATCH)], axis=1)
    return _mm(xcat, wcat_ref[...]) + b_ref[...]


def _up_conv(h, wcat_ref, b_ref):
    xe = jnp.concatenate([_shift_dn(h, BATCH), h, h], axis=1)
    oe = _mm(xe, wcat_ref[...]) + b_ref[...]
    xo = jnp.concatenate([h, h, _shift_up(h, BATCH)], axis=1)
    oo = _mm(xo, wcat_ref[...]) + b_ref[...]
    return oe, oo


def _call(body, out_shapes, args):
    return pl.pallas_call(
        body,
        out_shape=[jax.ShapeDtypeStruct(s, F32) for s in out_shapes],
    )(*args)


def _enc_level_body(he_ref, ho_ref, dw_ref, db_ref,
                    w10, b10, w20, b20, w11, b11, w21, b21,
                    w12, b12, w22, b22, o_ref):
    h = _down_conv(he_ref[...], ho_ref[...], dw_ref, db_ref)
    h = _res_block(h, w10, b10, w20, b20, 1)
    h = _res_block(h, w11, b11, w21, b21, 3)
    h = _res_block(h, w12, b12, w22, b22, 9)
    o_ref[...] = h


def _enc_level_out_body(he_ref, ho_ref, dw_ref, db_ref,
                        w10, b10, w20, b20, w11, b11, w21, b21,
                        w12, b12, w22, b22, ow_ref, ob_ref, o_ref):
    h = _down_conv(he_ref[...], ho_ref[...], dw_ref, db_ref)
    h = _res_block(h, w10, b10, w20, b20, 1)
    h = _res_block(h, w11, b11, w21, b21, 3)
    h = _res_block(h, w12, b12, w22, b22, 9)
    o_ref[...] = _conv3(h, ow_ref, ob_ref, 1)


def _lfq_body(xe_ref, wih_ref, wil_ref, bi_ref, cbt_ref, wo_ref, bo_ref,
              xq_ref, aux_ref):
    xe = xe_ref[...]
    z = _mm(xe, wih_ref[...]) + _mm(xe, wil_ref[...]) + bi_ref[...]
    rows = z.shape[0]
    q = jnp.where(z > 0.0, 1.0, -1.0)
    colmask = jax.lax.broadcasted_iota(jnp.int32, z.shape, 1) < CBD
    diff2 = jnp.where(colmask, (z - q) * (z - q), 0.0)
    commit = jnp.sum(jnp.sum(diff2, axis=1, keepdims=True), axis=0,
                     keepdims=True) / (rows * CBD)
    s = _mm(z, cbt_ref[...]) * 200.0
    m = jnp.max(s, axis=1, keepdims=True)
    e = jnp.exp(s - m)
    tot = jnp.sum(e, axis=1, keepdims=True)
    p = e / tot
    logp = jnp.log(jnp.clip(p, 1e-9, 1.0))
    pse = jnp.sum(jnp.sum(-p * logp, axis=1, keepdims=True), axis=0,
                  keepdims=True) / rows
    ap = jnp.sum(p, axis=0, keepdims=True) / rows
    ce = jnp.sum(-ap * jnp.log(jnp.clip(ap, 1e-9, 1.0)), axis=1,
                 keepdims=True)
    aux_ref[...] = 0.1 * (pse - ce) + 0.25 * commit
    xq_ref[...] = _mm(q, wo_ref[...]) + bo_ref[...]


def _dec_in_level_body(xq_ref, iw_ref, ib_ref,
                       w10, b10, w20, b20, w11, b11, w21, b21,
                       w12, b12, w22, b22, uw_ref, ub_ref,
                       oe_ref, oo_ref):
    h = jnp.maximum(_conv3(xq_ref[...], iw_ref, ib_ref, 1), 0.0)
    h = _res_block(h, w10, b10, w20, b20, 9)
    h = _res_block(h, w11, b11, w21, b21, 3)
    h = _res_block(h, w12, b12, w22, b22, 1)
    oe, oo = _up_conv(h, uw_ref, ub_ref)
    oe_ref[...] = oe
    oo_ref[...] = oo


def _dec_level_body(x_ref,
                    w10, b10, w20, b20, w11, b11, w21, b21,
                    w12, b12, w22, b22, uw_ref, ub_ref,
                    oe_ref, oo_ref):
    h = x_ref[...]
    h = _res_block(h, w10, b10, w20, b20, 9)
    h = _res_block(h, w11, b11, w21, b21, 3)
    h = _res_block(h, w12, b12, w22, b22, 1)
    oe, oo = _up_conv(h, uw_ref, ub_ref)
    oe_ref[...] = oe
    oo_ref[...] = oo


def _dec_out_body(x_ref, w1_ref, b1_ref, w2_ref, b2_ref, o_ref):
    h = jnp.maximum(_conv3(x_ref[...], w1_ref, b1_ref, 1), 0.0)
    o_ref[...] = _conv3(h, w2_ref, b2_ref, 1)


def _wcat(w, pad_in=0, pad_out=0):
    t = jnp.transpose(w, (2, 1, 0))
    if pad_in or pad_out:
        t = jnp.pad(t, ((0, 0), (0, pad_in), (0, pad_out)))
    return t.reshape(t.shape[0] * t.shape[1], t.shape[2])


def _bias(b):
    return b.reshape(1, -1)


def _split_phases(h, t):
    h4 = h.reshape(t // 2, 2, BATCH, CH)
    return (h4[:, 0].reshape(t // 2 * BATCH, CH),
            h4[:, 1].reshape(t // 2 * BATCH, CH))


def _interleave(oe, oo, t):
    st = jnp.stack([oe.reshape(t, BATCH, CH), oo.reshape(t, BATCH, CH)],
                   axis=1)
    return st.reshape(2 * t * BATCH, CH)


_CBT = np.zeros((128, NCODE), np.float32)
_CBT[:CBD, :] = (((np.arange(NCODE)[None, :] >> np.arange(CBD)[:, None]) & 1)
                 .astype(np.float32)) * 2.0 - 1.0


def _res_args(p, side, lvl):
    out = []
    for d in range(3):
        out += [_wcat(p['%s_res%d_%d_w1' % (side, lvl, d)]),
                _bias(p['%s_res%d_%d_b1' % (side, lvl, d)]),
                _wcat(p['%s_res%d_%d_w2' % (side, lvl, d)]),
                _bias(p['%s_res%d_%d_b2' % (side, lvl, d)])]
    return out


def kernel(x, params):
    p = params
    h0 = jax.nn.relu(jax.lax.conv_general_dilated(
        x.transpose(0, 2, 1), p['enc_in_w'], (1,), [(1, 1)],
        dimension_numbers=('NCH', 'OIH', 'NCH'))
        + p['enc_in_b'][None, :, None])
    h = jnp.transpose(h0, (2, 0, 1)).reshape(SEQ * BATCH, CH)
    t = SEQ
    for lvl in range(3):
        he, ho = _split_phases(h, t)
        t //= 2
        args = [he, ho, _wcat(p['enc_down%d_w' % lvl]),
                _bias(p['enc_down%d_b' % lvl])] + _res_args(p, 'enc', lvl)
        if lvl < 2:
            (h,) = _call(_enc_level_body, [(t * BATCH, CH)], args)
        else:
            args += [_wcat(p['enc_out_w']), _bias(p['enc_out_b'])]
            (h,) = _call(_enc_level_out_body, [(t * BATCH, CH)], args)

    wi = jnp.pad(p['lfq_in_w'].T, ((0, 0), (0, 128 - CBD)))
    wih = wi.astype(jnp.bfloat16).astype(F32)
    wil = (wi - wih).astype(jnp.bfloat16).astype(F32)
    bi = jnp.pad(_bias(p['lfq_in_b']), ((0, 0), (0, 128 - CBD)))
    wo = jnp.pad(p['lfq_out_w'].T, ((0, 128 - CBD), (0, 0)))
    xq, aux = _call(_lfq_body, [(t * BATCH, CH), (1, 1)],
                    [h, wih, wil, bi, jnp.asarray(_CBT), wo,
                     _bias(p['lfq_out_b'])])

    args = [xq, _wcat(p['dec_in_w']), _bias(p['dec_in_b'])] \
        + _res_args(p, 'dec', 0) \
        + [_wcat(p['dec_up0_w']), _bias(p['dec_up0_b'])]
    oe, oo = _call(_dec_in_level_body, [(t * BATCH, CH)] * 2, args)
    h = _interleave(oe, oo, t)
    t *= 2
    for lvl in (1, 2):
        args = [h] + _res_args(p, 'dec', lvl) \
            + [_wcat(p['dec_up%d_w' % lvl]), _bias(p['dec_up%d_b' % lvl])]
        oe, oo = _call(_dec_level_body, [(t * BATCH, CH)] * 2, args)
        h = _interleave(oe, oo, t)
        t *= 2

    w2 = _wcat(p['dec_out2_w'], pad_out=IN_WP - IN_W)
    b2 = jnp.pad(_bias(p['dec_out2_b']), ((0, 0), (0, IN_WP - IN_W)))
    (y,) = _call(_dec_out_body, [(t * BATCH, IN_WP)],
                 [h, _wcat(p['dec_out1_w']), _bias(p['dec_out1_b']),
                  w2, b2])

    x_out = y[:, :IN_W].reshape(SEQ, BATCH, IN_W).transpose(1, 0, 2)
    return x_out, aux.reshape(())
